# baseline (speedup 1.0000x reference)
import functools

import jax
import jax.numpy as jnp
from jax import lax
from jax.experimental import pallas as pl
from jax.experimental.pallas import tpu as pltpu
from jax.experimental.pallas import tpu_sc as plsc

DIM = 128
CHUNK = 64
NBUF = 12
LA = 6


def _sc_gather(table, idx3d):
    info = plsc.get_sparse_core_info()
    nc, ns = info.num_cores, info.num_subcores
    nw = nc * ns
    chunks_per_w = idx3d.shape[0] // (nw * CHUNK)
    rows_per_w = chunks_per_w * CHUNK
    total = nw * rows_per_w

    mesh = plsc.VectorSubcoreMesh(core_axis_name="c", subcore_axis_name="s")

    @functools.partial(
        pl.kernel,
        mesh=mesh,
        out_type=jax.ShapeDtypeStruct((total, DIM), jnp.float32),
        scratch_types=[
            pltpu.VMEM((chunks_per_w * CHUNK,), jnp.int32),
        ]
        + [pltpu.VMEM((CHUNK, DIM), jnp.float32)] * NBUF
        + [pltpu.SemaphoreType.DMA] * (2 * NBUF),
    )
    def k(table_hbm, idx_hbm, out_hbm, idx_v, *scratch):
        rows = scratch[:NBUF]
        gsem = scratch[NBUF : 2 * NBUF]
        osem = scratch[2 * NBUF :]
        wid = lax.axis_index("s") * nc + lax.axis_index("c")
        obase = wid * rows_per_w
        pltpu.sync_copy(idx_hbm.at[pl.ds(wid * rows_per_w, rows_per_w)], idx_v)

        def gstart(j, b):
            pltpu.async_copy(
                table_hbm.at[idx_v.at[pl.ds(j * CHUNK, CHUNK)]], rows[b], gsem[b]
            )

        def step(j, b, do_owait, do_gstart):
            pltpu.make_async_copy(
                table_hbm.at[idx_v.at[pl.ds(j * CHUNK, CHUNK)]], rows[b], gsem[b]
            ).wait()
            pltpu.async_copy(
                rows[b], out_hbm.at[pl.ds(obase + j * CHUNK, CHUNK)], osem[b]
            )
            if do_gstart:
                jn = j + LA
                bn = (b + LA) % NBUF
                if do_owait:
                    pltpu.make_async_copy(
                        rows[bn], out_hbm.at[pl.ds(obase, CHUNK)], osem[bn]
                    ).wait()
                pltpu.async_copy(
                    table_hbm.at[idx_v.at[pl.ds(jn * CHUNK, CHUNK)]], rows[bn], gsem[bn]
                )

        n = chunks_per_w
        for j in range(LA):
            gstart(j, j % NBUF)
        for j in range(NBUF - LA):
            step(j, j % NBUF, False, True)
        nmain = n - NBUF
        nblocks = nmain // NBUF

        def body(g, carry):
            j0 = (NBUF - LA) + g * NBUF
            for t in range(NBUF):
                step(j0 + t, (NBUF - LA + t) % NBUF, True, True)
            return carry

        lax.fori_loop(0, nblocks, body, 0)
        for j in range(NBUF - LA + nblocks * NBUF, n - LA):
            step(j, j % NBUF, True, True)
        for j in range(n - LA, n):
            step(j, j % NBUF, False, False)
        for j in range(n - NBUF, n):
            b = j % NBUF
            pltpu.make_async_copy(
                rows[b], out_hbm.at[pl.ds(obase, CHUNK)], osem[b]
            ).wait()

    return k(table, idx3d)


def kernel(x, weight):
    flat = x.T.reshape(-1).astype(jnp.int32)
    out = _sc_gather(weight, flat)
    return out.reshape(x.shape[1], x.shape[0], DIM).transpose(1, 0, 2)


# speedup vs baseline: 1.0029x; 1.0029x over previous
import functools

import jax
import jax.numpy as jnp
from jax import lax
from jax.experimental import pallas as pl
from jax.experimental.pallas import tpu as pltpu
from jax.experimental.pallas import tpu_sc as plsc

DIM = 128
CHUNK = 128
NBUF = 7
LA = 5


def _sc_gather(table, idx3d):
    info = plsc.get_sparse_core_info()
    nc, ns = info.num_cores, info.num_subcores
    nw = nc * ns
    chunks_per_w = idx3d.shape[0] // (nw * CHUNK)
    rows_per_w = chunks_per_w * CHUNK
    total = nw * rows_per_w

    mesh = plsc.VectorSubcoreMesh(core_axis_name="c", subcore_axis_name="s")

    @functools.partial(
        pl.kernel,
        mesh=mesh,
        out_type=jax.ShapeDtypeStruct((total, DIM), jnp.float32),
        scratch_types=[
            pltpu.VMEM((chunks_per_w * CHUNK,), jnp.int32),
        ]
        + [pltpu.VMEM((CHUNK, DIM), jnp.float32)] * NBUF
        + [pltpu.SemaphoreType.DMA] * (2 * NBUF),
    )
    def k(table_hbm, idx_hbm, out_hbm, idx_v, *scratch):
        rows = scratch[:NBUF]
        gsem = scratch[NBUF : 2 * NBUF]
        osem = scratch[2 * NBUF :]
        wid = lax.axis_index("s") * nc + lax.axis_index("c")
        obase = wid * rows_per_w
        pltpu.sync_copy(idx_hbm.at[pl.ds(wid * rows_per_w, rows_per_w)], idx_v)

        def gstart(j, b):
            pltpu.async_copy(
                table_hbm.at[idx_v.at[pl.ds(j * CHUNK, CHUNK)]], rows[b], gsem[b]
            )

        def step(j, b, do_owait, do_gstart):
            pltpu.make_async_copy(
                table_hbm.at[idx_v.at[pl.ds(j * CHUNK, CHUNK)]], rows[b], gsem[b]
            ).wait()
            pltpu.async_copy(
                rows[b], out_hbm.at[pl.ds(obase + j * CHUNK, CHUNK)], osem[b]
            )
            if do_gstart:
                jn = j + LA
                bn = (b + LA) % NBUF
                if do_owait:
                    pltpu.make_async_copy(
                        rows[bn], out_hbm.at[pl.ds(obase, CHUNK)], osem[bn]
                    ).wait()
                pltpu.async_copy(
                    table_hbm.at[idx_v.at[pl.ds(jn * CHUNK, CHUNK)]], rows[bn], gsem[bn]
                )

        n = chunks_per_w
        for j in range(LA):
            gstart(j, j % NBUF)
        for j in range(NBUF - LA):
            step(j, j % NBUF, False, True)
        nmain = n - NBUF
        nblocks = nmain // NBUF

        def body(g, carry):
            j0 = (NBUF - LA) + g * NBUF
            for t in range(NBUF):
                step(j0 + t, (NBUF - LA + t) % NBUF, True, True)
            return carry

        lax.fori_loop(0, nblocks, body, 0)
        for j in range(NBUF - LA + nblocks * NBUF, n - LA):
            step(j, j % NBUF, True, True)
        for j in range(n - LA, n):
            step(j, j % NBUF, False, False)
        for j in range(n - NBUF, n):
            b = j % NBUF
            pltpu.make_async_copy(
                rows[b], out_hbm.at[pl.ds(obase, CHUNK)], osem[b]
            ).wait()

    return k(table, idx3d)


def kernel(x, weight):
    flat = x.T.reshape(-1).astype(jnp.int32)
    out = _sc_gather(weight, flat)
    return out.reshape(x.shape[1], x.shape[0], DIM).transpose(1, 0, 2)


# speedup vs baseline: 1.0033x; 1.0004x over previous
import functools

import jax
import jax.numpy as jnp
from jax import lax
from jax.experimental import pallas as pl
from jax.experimental.pallas import tpu as pltpu
from jax.experimental.pallas import tpu_sc as plsc

DIM = 128
CHUNK = 128
NBUF = 7
LA = 4


def _sc_gather(table, idx3d):
    info = plsc.get_sparse_core_info()
    nc, ns = info.num_cores, info.num_subcores
    nw = nc * ns
    chunks_per_w = idx3d.shape[0] // (nw * CHUNK)
    rows_per_w = chunks_per_w * CHUNK
    total = nw * rows_per_w

    mesh = plsc.VectorSubcoreMesh(core_axis_name="c", subcore_axis_name="s")

    @functools.partial(
        pl.kernel,
        mesh=mesh,
        out_type=jax.ShapeDtypeStruct((total, DIM), jnp.float32),
        scratch_types=[
            pltpu.VMEM((chunks_per_w * CHUNK,), jnp.int32),
        ]
        + [pltpu.VMEM((CHUNK, DIM), jnp.float32)] * NBUF
        + [pltpu.SemaphoreType.DMA] * (2 * NBUF),
    )
    def k(table_hbm, idx_hbm, out_hbm, idx_v, *scratch):
        rows = scratch[:NBUF]
        gsem = scratch[NBUF : 2 * NBUF]
        osem = scratch[2 * NBUF :]
        wid = lax.axis_index("s") * nc + lax.axis_index("c")
        obase = wid * rows_per_w
        pltpu.sync_copy(idx_hbm.at[pl.ds(wid * rows_per_w, rows_per_w)], idx_v)

        def gstart(j, b):
            pltpu.async_copy(
                table_hbm.at[idx_v.at[pl.ds(j * CHUNK, CHUNK)]], rows[b], gsem[b]
            )

        def step(j, b, do_owait, do_gstart):
            pltpu.make_async_copy(
                table_hbm.at[idx_v.at[pl.ds(j * CHUNK, CHUNK)]], rows[b], gsem[b]
            ).wait()
            pltpu.async_copy(
                rows[b], out_hbm.at[pl.ds(obase + j * CHUNK, CHUNK)], osem[b]
            )
            if do_gstart:
                jn = j + LA
                bn = (b + LA) % NBUF
                if do_owait:
                    pltpu.make_async_copy(
                        rows[bn], out_hbm.at[pl.ds(obase, CHUNK)], osem[bn]
                    ).wait()
                pltpu.async_copy(
                    table_hbm.at[idx_v.at[pl.ds(jn * CHUNK, CHUNK)]], rows[bn], gsem[bn]
                )

        n = chunks_per_w
        for j in range(LA):
            gstart(j, j % NBUF)
        for j in range(NBUF - LA):
            step(j, j % NBUF, False, True)
        nmain = n - NBUF
        nblocks = nmain // NBUF

        def body(g, carry):
            j0 = (NBUF - LA) + g * NBUF
            for t in range(NBUF):
                step(j0 + t, (NBUF - LA + t) % NBUF, True, True)
            return carry

        lax.fori_loop(0, nblocks, body, 0)
        for j in range(NBUF - LA + nblocks * NBUF, n - LA):
            step(j, j % NBUF, True, True)
        for j in range(n - LA, n):
            step(j, j % NBUF, False, False)
        for j in range(n - NBUF, n):
            b = j % NBUF
            pltpu.make_async_copy(
                rows[b], out_hbm.at[pl.ds(obase, CHUNK)], osem[b]
            ).wait()

    return k(table, idx3d)


def kernel(x, weight):
    flat = x.T.reshape(-1).astype(jnp.int32)
    out = _sc_gather(weight, flat)
    return out.reshape(x.shape[1], x.shape[0], DIM).transpose(1, 0, 2)
